# Initial kernel scaffold; baseline (speedup 1.0000x reference)
import jax
import jax.numpy as jnp
from jax.experimental import pallas as pl


def kernel(h, edge_index, e, params):
    raise NotImplementedError("write your pallas kernel here")


# speedup vs baseline: 1.0010x; 1.0010x over previous
import functools

import jax
import jax.numpy as jnp
from jax.experimental import pallas as pl
from jax.experimental.pallas import tpu as pltpu

N = 15279
E = 488928
HID = 128
NP = 15360


def _embed_kernel(h_ref, emb_ref, filt_ref, y_ref):
    h = h_ref[...]
    lanes = jax.lax.broadcasted_iota(jnp.int32, (h.shape[0], 128), 1)
    oh = (h == lanes).astype(jnp.float32)
    x = jnp.dot(oh, emb_ref[...], preferred_element_type=jnp.float32)
    sig = jax.nn.sigmoid(filt_ref[...])
    y_ref[...] = x * sig


def _embed(h_pad, emb, filt_pad):
    return pl.pallas_call(
        _embed_kernel,
        out_shape=jax.ShapeDtypeStruct((NP, HID), jnp.float32),
    )(h_pad, emb, filt_pad)


def _dense_kernel(has_next, y_ref, agg_ref, deg_ref, wt_ref, wb_ref, b_ref,
                  g_ref, beta_ref, w0_ref, b0_ref, w1_ref, b1_ref, w2_ref,
                  b2_ref, nfilt_ref, logits_ref, ynext_ref=None):
    y = y_ref[...]
    deg = jnp.clip(deg_ref[...], 1.0, None)
    mean_n = agg_ref[...] * (1.0 / deg)
    t = (jnp.dot(y, wt_ref[...], preferred_element_type=jnp.float32)
         + jnp.dot(mean_n, wb_ref[...], preferred_element_type=jnp.float32)
         + b_ref[...])
    t = jnp.maximum(t, 0.0)
    rows = jax.lax.broadcasted_iota(jnp.int32, (NP, 1), 0)
    mask = (rows < N).astype(jnp.float32)
    tm = t * mask
    mu = jnp.sum(tm, axis=0, keepdims=True) / N
    ex2 = jnp.sum(tm * tm, axis=0, keepdims=True) / N
    var = ex2 - mu * mu
    tn = (t - mu) * jax.lax.rsqrt(var + 1e-5) * g_ref[...] + beta_ref[...]
    out = y + tn
    z = jnp.maximum(jnp.dot(out, w0_ref[...], preferred_element_type=jnp.float32) + b0_ref[...], 0.0)
    z = jnp.maximum(jnp.dot(z, w1_ref[...], preferred_element_type=jnp.float32) + b1_ref[...], 0.0)
    logits_ref[...] = jnp.dot(z, w2_ref[...], preferred_element_type=jnp.float32) + b2_ref[...]
    if has_next:
        ynext_ref[...] = out * jax.nn.sigmoid(nfilt_ref[...])


def _pad_w(w, rows=HID, cols=HID):
    out = jnp.zeros((rows, cols), jnp.float32)
    return out.at[: w.shape[0], : w.shape[1]].set(w)


def _pad_b(b, cols=HID):
    out = jnp.zeros((1, cols), jnp.float32)
    return out.at[0, : b.shape[0]].set(b)


def _dense_stage(y, agg, deg, lp, mp, next_filt_pad):
    has_next = next_filt_pad is not None
    wt = lp['W'][:HID]
    wb = lp['W'][HID:]
    b = lp['b'][None, :]
    g = lp['gamma'][None, :]
    beta = lp['beta'][None, :]
    w0 = _pad_w(mp['Ws'][0])
    b0 = _pad_b(mp['bs'][0])
    w1 = _pad_w(mp['Ws'][1])
    b1 = _pad_b(mp['bs'][1])
    w2 = _pad_w(mp['Ws'][2])
    b2 = _pad_b(mp['bs'][2])
    nfilt = next_filt_pad if has_next else jnp.zeros((NP, 1), jnp.float32)
    out_shape = [jax.ShapeDtypeStruct((NP, HID), jnp.float32)]
    if has_next:
        out_shape.append(jax.ShapeDtypeStruct((NP, HID), jnp.float32))
    res = pl.pallas_call(
        functools.partial(_dense_kernel, has_next),
        out_shape=out_shape,
    )(y, agg, deg, wt, wb, b, g, beta, w0, b0, w1, b1, w2, b2, nfilt)
    if has_next:
        return res[0], res[1]
    return res[0], None


def _seg_agg(y_pad, src, dst):
    agg = jax.ops.segment_sum(y_pad[src], dst, num_segments=N)
    return jnp.pad(agg, ((0, NP - N), (0, 0)))


def kernel(h, edge_index, e, params):
    src = edge_index[0]
    dst = edge_index[1]
    h_pad = jnp.pad(h.astype(jnp.int32), (0, NP - N)).reshape(NP, 1)

    def pad_filt(f):
        return jnp.pad(f, ((0, NP - N), (0, 0)))

    deg = jax.ops.segment_sum(jnp.ones((E,), jnp.float32), dst, num_segments=N)
    deg_pad = jnp.pad(deg, (0, NP - N)).reshape(NP, 1)

    y1 = _embed(h_pad, params['emb'], pad_filt(params['cvx_filter']))
    agg1 = _seg_agg(y1, src, dst)
    logits1, y2 = _dense_stage(y1, agg1, deg_pad, params['cvx_layers'][3],
                               params['mlp_cvx'], pad_filt(params['cov_filter']))
    agg2 = _seg_agg(y2, src, dst)
    logits2, y3 = _dense_stage(y2, agg2, deg_pad, params['cov_layers'][3],
                               params['mlp_cov'], pad_filt(params['normal_filter']))
    agg3 = _seg_agg(y3, src, dst)
    logits3, _ = _dense_stage(y3, agg3, deg_pad, params['normal_layers'][3],
                              params['mlp_normal'], None)
    return jnp.concatenate([logits1[:N, :6], logits2[:N, :6], logits3[:N, :6]], axis=0)


# speedup vs baseline: 2.9514x; 2.9485x over previous
import functools

import jax
import jax.numpy as jnp
from jax import lax
from jax.experimental import pallas as pl
from jax.experimental.pallas import tpu as pltpu
from jax.experimental.pallas import tpu_sc as plsc

N = 15279
E = 488928
HID = 128
NP = 15360
NPT = 480
NPT_A = 488
NW = 32
NC = 2
NS = 16

EPAD = 491520
FCH = 4096
NFCH = EPAD // FCH
RING = 8192
FB = 4096
GR = 256
CAP = 32768
K = 128
DSENT = 480
BIGDST = 1 << 20

_mesh = plsc.VectorSubcoreMesh(core_axis_name="c", subcore_axis_name="s")
_sc_params = pltpu.CompilerParams(needs_layout_passes=False)


def _bcast_lane(v, lane):
    idx = jnp.full((16, 1), lane, jnp.int32)
    dnums = lax.GatherDimensionNumbers(
        offset_dims=(), collapsed_slice_dims=(0,), start_index_map=(0,))
    return lax.gather(v, idx, dnums, slice_sizes=(1,),
                      mode=lax.GatherScatterMode.PROMISE_IN_BOUNDS)


def _bcast_last(v):
    return _bcast_lane(v, 15)


def _filter_body(src_hbm, dst_hbm, edges_hbm, cnt_hbm,
                 sbuf0, sbuf1, dbuf0, dbuf1, ring, cnt_v,
                 sem_s0, sem_s1, sem_d0, sem_d1):
    wid = lax.axis_index("s") * NC + lax.axis_index("c")
    lo = wid * NPT
    hi = lo + NPT
    out_base = wid * CAP
    iota = lax.iota(jnp.int32, 16)
    sent = ((wid * 16 + iota) << 9) | DSENT

    bufs = [(sbuf0, dbuf0, sem_s0, sem_d0), (sbuf1, dbuf1, sem_s1, sem_d1)]

    def issue(g, p):
        sb, db, ss, ds = bufs[p]
        pltpu.async_copy(src_hbm.at[pl.ds(g * FCH, FCH)], sb, ss)
        pltpu.async_copy(dst_hbm.at[pl.ds(g * FCH, FCH)], db, ds)

    def wait(p):
        sb, db, ss, ds = bufs[p]
        pltpu.make_async_copy(src_hbm.at[pl.ds(0, FCH)], sb, ss).wait()
        pltpu.make_async_copy(dst_hbm.at[pl.ds(0, FCH)], db, ds).wait()

    issue(0, 0)
    issue(1, 1)

    def process_chunk(g, p, carry):
        off, flushed = carry
        sb, db, _, _ = bufs[p]

        def vreg_body(j, off):
            s = sb[pl.ds(j * 16, 16)]
            d = db[pl.ds(j * 16, 16)]
            m = (d >= lo) & (d < hi) & (off < CAP - RING)
            m01 = jnp.where(m, 1, 0).astype(jnp.int32)
            cs = plsc.cumsum(m01)
            pos = (off + cs - 1) & (RING - 1)
            pk = (s << 9) | (d - lo)
            plsc.store_scatter(ring, [pos], pk, mask=m)
            return off + _bcast_last(cs)

        off = lax.fori_loop(0, FCH // 16, vreg_body, off)

        @pl.when(g + 2 < NFCH)
        def _():
            issue(g + 2, p)

        total = jnp.max(off)
        do_flush = total - flushed >= FB

        @pl.when(do_flush)
        def _():
            half = (flushed // FB) % 2
            pltpu.sync_copy(ring.at[pl.ds(pl.multiple_of(half * FB, FB), FB)],
                            edges_hbm.at[pl.ds(pl.multiple_of(out_base + flushed, FB), FB)])

        flushed = jnp.where(do_flush, flushed + FB, flushed)
        return off, flushed

    carry = (jnp.zeros((16,), jnp.int32), jnp.int32(0))

    def outer(i, carry):
        g = i * 2
        wait(0)
        carry = process_chunk(g, 0, carry)
        wait(1)
        carry = process_chunk(g + 1, 1, carry)
        return carry

    off, flushed = lax.fori_loop(0, NFCH // 2, outer, carry)

    def pad_body(k, _):
        pos = (off + k * 16 + iota) & (RING - 1)
        plsc.store_scatter(ring, [pos], sent)
        return 0

    lax.fori_loop(0, GR // 16, pad_body, 0)

    total = jnp.max(off)
    padded = ((total + GR - 1) // GR) * GR

    def drain_cond(fl):
        return fl < padded

    def drain(fl):
        pltpu.sync_copy(ring.at[pl.ds(pl.multiple_of(fl % RING, GR), GR)],
                        edges_hbm.at[pl.ds(pl.multiple_of(out_base + fl, GR), GR)])
        return fl + GR

    lax.while_loop(drain_cond, drain, flushed)

    cnt_v[...] = jnp.zeros((16,), jnp.int32) + padded
    pltpu.sync_copy(cnt_v, cnt_hbm.at[wid])


def _filter_edges(src_pad, dst_pad):
    f = pl.kernel(
        _filter_body,
        out_type=[
            jax.ShapeDtypeStruct((NW * CAP,), jnp.int32),
            jax.ShapeDtypeStruct((NW, 16), jnp.int32),
        ],
        mesh=_mesh,
        scratch_types=[
            pltpu.VMEM((FCH,), jnp.int32),
            pltpu.VMEM((FCH,), jnp.int32),
            pltpu.VMEM((FCH,), jnp.int32),
            pltpu.VMEM((FCH,), jnp.int32),
            pltpu.VMEM((RING,), jnp.int32),
            pltpu.VMEM((16,), jnp.int32),
            pltpu.SemaphoreType.DMA,
            pltpu.SemaphoreType.DMA,
            pltpu.SemaphoreType.DMA,
            pltpu.SemaphoreType.DMA,
        ],
        compiler_params=_sc_params,
    )
    return f(src_pad, dst_pad)


def _agg_body(with_deg, *refs):
    if with_deg:
        (y_hbm, edges_hbm, cnt_hbm, agg_hbm, deg_hbm,
         pk0, pk1, si0, si1, di0, di1, rows0, rows1, agg_v, deg_v,
         cnt_v, sem_p0, sem_p1, sem_g0, sem_g1) = refs
    else:
        (y_hbm, edges_hbm, cnt_hbm, agg_hbm,
         pk0, pk1, si0, si1, di0, di1, rows0, rows1, agg_v,
         cnt_v, sem_p0, sem_p1, sem_g0, sem_g1) = refs
        deg_hbm = deg_v = None

    wid = lax.axis_index("s") * NC + lax.axis_index("c")
    out_base = wid * CAP
    zero16 = jnp.zeros((16,), jnp.float32)
    ones16 = zero16 + 1.0
    iota = lax.iota(jnp.int32, 16)
    col_idx = [iota + c * 16 for c in range(8)]

    def zrow(i, _):
        for c in range(8):
            agg_v[i, pl.ds(c * 16, 16)] = zero16
        return 0

    lax.fori_loop(0, NPT_A, zrow, 0)

    if with_deg:
        def zdeg(i, _):
            deg_v[pl.ds(i * 16, 16)] = zero16
            return 0
        lax.fori_loop(0, NPT_A, zdeg, 0)

    pltpu.sync_copy(cnt_hbm.at[wid], cnt_v)
    n_chunks = jnp.max(cnt_v[...]) >> 7

    bufs = [(pk0, si0, di0, rows0, sem_p0, sem_g0),
            (pk1, si1, di1, rows1, sem_p1, sem_g1)]

    def issue_pk(g, p):
        pk = bufs[p][0]
        sp = bufs[p][4]
        pltpu.async_copy(
            edges_hbm.at[pl.ds(pl.multiple_of(out_base + g * K, K), K)], pk, sp)

    @pl.when(n_chunks > 0)
    def _():
        issue_pk(0, 0)

    @pl.when(n_chunks > 1)
    def _():
        issue_pk(1, 1)

    def chunk(g, p):
        pk, si, di, rows, sp, sg = bufs[p]

        @pl.when(g < n_chunks)
        def _():
            pltpu.make_async_copy(edges_hbm.at[pl.ds(0, K)], pk, sp).wait()
            for j in range(K // 16):
                v = pk[pl.ds(j * 16, 16)]
                si[pl.ds(j * 16, 16)] = v >> 9
                di[pl.ds(j * 16, 16)] = v & (512 - 1)

            @pl.when(g + 2 < n_chunks)
            def _():
                issue_pk(g + 2, p)

            pltpu.async_copy(y_hbm.at[si], rows, sg).wait()

            def blk_body(blk, _):
                dvec = di[pl.ds(blk * 16, 16)]
                for e in range(16):
                    db = _bcast_lane(dvec, e)
                    ei = blk * 16 + e
                    for c in range(8):
                        val = rows[ei, pl.ds(c * 16, 16)]
                        plsc.addupdate_scatter(agg_v, [db, col_idx[c]], val)
                    if with_deg:
                        plsc.addupdate_scatter(deg_v, [db * 16 + iota], ones16)
                return 0

            lax.fori_loop(0, K // 16, blk_body, 0)

    def outer(i, _):
        chunk(i * 2, 0)
        chunk(i * 2 + 1, 1)
        return 0

    lax.fori_loop(0, (n_chunks + 1) // 2, outer, 0)

    pltpu.sync_copy(agg_v.at[pl.ds(0, NPT)], agg_hbm.at[pl.ds(wid * NPT, NPT)])
    if with_deg:
        pltpu.sync_copy(deg_v.at[pl.ds(0, NPT * 16)],
                        deg_hbm.at[pl.ds(wid * NPT * 16, NPT * 16)])


def _agg(y, edges_p, cnt, with_deg):
    out_type = [jax.ShapeDtypeStruct((NP, HID), jnp.float32)]
    if with_deg:
        out_type.append(jax.ShapeDtypeStruct((NP * 16,), jnp.float32))
    scratch = [
        pltpu.VMEM((K,), jnp.int32),
        pltpu.VMEM((K,), jnp.int32),
        pltpu.VMEM((K,), jnp.int32),
        pltpu.VMEM((K,), jnp.int32),
        pltpu.VMEM((K,), jnp.int32),
        pltpu.VMEM((K,), jnp.int32),
        pltpu.VMEM((K, HID), jnp.float32),
        pltpu.VMEM((K, HID), jnp.float32),
        pltpu.VMEM((NPT_A, HID), jnp.float32),
    ]
    if with_deg:
        scratch.append(pltpu.VMEM((NPT_A * 16,), jnp.float32))
    scratch.append(pltpu.VMEM((16,), jnp.int32))
    scratch += [pltpu.SemaphoreType.DMA] * 4
    f = pl.kernel(
        functools.partial(_agg_body, with_deg),
        out_type=out_type,
        mesh=_mesh,
        scratch_types=scratch,
        compiler_params=_sc_params,
    )
    return f(y, edges_p, cnt)


def _embed_kernel(h_ref, emb_ref, filt_ref, y_ref):
    h = h_ref[...]
    lanes = jax.lax.broadcasted_iota(jnp.int32, (h.shape[0], 128), 1)
    oh = (h == lanes).astype(jnp.float32)
    x = jnp.dot(oh, emb_ref[...], preferred_element_type=jnp.float32)
    sig = jax.nn.sigmoid(filt_ref[...])
    y_ref[...] = x * sig


def _embed(h_pad, emb, filt_pad):
    return pl.pallas_call(
        _embed_kernel,
        out_shape=jax.ShapeDtypeStruct((NP, HID), jnp.float32),
    )(h_pad, emb, filt_pad)


def _dense_kernel(has_next, y_ref, agg_ref, deg_ref, wt_ref, wb_ref, b_ref,
                  g_ref, beta_ref, w0_ref, b0_ref, w1_ref, b1_ref, w2_ref,
                  b2_ref, nfilt_ref, logits_ref, ynext_ref=None):
    y = y_ref[...]
    deg = jnp.clip(deg_ref[...], 1.0, None)
    mean_n = agg_ref[...] * (1.0 / deg)
    t = (jnp.dot(y, wt_ref[...], preferred_element_type=jnp.float32)
         + jnp.dot(mean_n, wb_ref[...], preferred_element_type=jnp.float32)
         + b_ref[...])
    t = jnp.maximum(t, 0.0)
    rows = jax.lax.broadcasted_iota(jnp.int32, (NP, 1), 0)
    mask = (rows < N).astype(jnp.float32)
    tm = t * mask
    mu = jnp.sum(tm, axis=0, keepdims=True) / N
    ex2 = jnp.sum(tm * tm, axis=0, keepdims=True) / N
    var = ex2 - mu * mu
    tn = (t - mu) * jax.lax.rsqrt(var + 1e-5) * g_ref[...] + beta_ref[...]
    out = y + tn
    z = jnp.maximum(jnp.dot(out, w0_ref[...], preferred_element_type=jnp.float32) + b0_ref[...], 0.0)
    z = jnp.maximum(jnp.dot(z, w1_ref[...], preferred_element_type=jnp.float32) + b1_ref[...], 0.0)
    logits_ref[...] = jnp.dot(z, w2_ref[...], preferred_element_type=jnp.float32) + b2_ref[...]
    if has_next:
        ynext_ref[...] = out * jax.nn.sigmoid(nfilt_ref[...])


def _pad_w(w, rows=HID, cols=HID):
    out = jnp.zeros((rows, cols), jnp.float32)
    return out.at[: w.shape[0], : w.shape[1]].set(w)


def _pad_b(b, cols=HID):
    out = jnp.zeros((1, cols), jnp.float32)
    return out.at[0, : b.shape[0]].set(b)


def _dense_stage(y, agg, deg, lp, mp, next_filt_pad):
    has_next = next_filt_pad is not None
    wt = lp['W'][:HID]
    wb = lp['W'][HID:]
    b = lp['b'][None, :]
    g = lp['gamma'][None, :]
    beta = lp['beta'][None, :]
    w0 = _pad_w(mp['Ws'][0])
    b0 = _pad_b(mp['bs'][0])
    w1 = _pad_w(mp['Ws'][1])
    b1 = _pad_b(mp['bs'][1])
    w2 = _pad_w(mp['Ws'][2])
    b2 = _pad_b(mp['bs'][2])
    nfilt = next_filt_pad if has_next else jnp.zeros((NP, 1), jnp.float32)
    out_shape = [jax.ShapeDtypeStruct((NP, HID), jnp.float32)]
    if has_next:
        out_shape.append(jax.ShapeDtypeStruct((NP, HID), jnp.float32))
    res = pl.pallas_call(
        functools.partial(_dense_kernel, has_next),
        out_shape=out_shape,
    )(y, agg, deg, wt, wb, b, g, beta, w0, b0, w1, b1, w2, b2, nfilt)
    if has_next:
        return res[0], res[1]
    return res[0], None


def kernel(h, edge_index, e, params):
    src = edge_index[0].astype(jnp.int32)
    dst = edge_index[1].astype(jnp.int32)
    src_pad = jnp.pad(src, (0, EPAD - E))
    dst_pad = jnp.pad(dst, (0, EPAD - E), constant_values=BIGDST)
    h_pad = jnp.pad(h.astype(jnp.int32), (0, NP - N)).reshape(NP, 1)

    def pad_filt(f):
        return jnp.pad(f, ((0, NP - N), (0, 0)))

    edges_p, cnt = _filter_edges(src_pad, dst_pad)

    y1 = _embed(h_pad, params['emb'], pad_filt(params['cvx_filter']))
    agg1, deg16 = _agg(y1, edges_p, cnt, True)
    deg_pad = deg16.reshape(NP, 16)[:, :1]
    logits1, y2 = _dense_stage(y1, agg1, deg_pad, params['cvx_layers'][3],
                               params['mlp_cvx'], pad_filt(params['cov_filter']))
    agg2 = _agg(y2, edges_p, cnt, False)[0]
    logits2, y3 = _dense_stage(y2, agg2, deg_pad, params['cov_layers'][3],
                               params['mlp_cov'], pad_filt(params['normal_filter']))
    agg3 = _agg(y3, edges_p, cnt, False)[0]
    logits3, _ = _dense_stage(y3, agg3, deg_pad, params['normal_layers'][3],
                              params['mlp_normal'], None)
    return jnp.concatenate([logits1[:N, :6], logits2[:N, :6], logits3[:N, :6]], axis=0)


# speedup vs baseline: 4.0347x; 1.3670x over previous
import functools

import jax
import jax.numpy as jnp
from jax import lax
from jax.experimental import pallas as pl
from jax.experimental.pallas import tpu as pltpu
from jax.experimental.pallas import tpu_sc as plsc

N = 15279
E = 488928
HID = 128
NP = 15360
NPT = 480
NPT_A = 488
NW = 32
NC = 2
NS = 16

EPAD = 491520
FCH = 4096
NFCH = EPAD // FCH
RING = 8192
FB = 4096
GR = 256
CAP = 32768
K = 128
DSENT = 480
BIGDST = 1 << 20

_mesh = plsc.VectorSubcoreMesh(core_axis_name="c", subcore_axis_name="s")
_sc_params = pltpu.CompilerParams(needs_layout_passes=False)


def _bcast_lane(v, lane):
    idx = jnp.full((16, 1), lane, jnp.int32)
    dnums = lax.GatherDimensionNumbers(
        offset_dims=(), collapsed_slice_dims=(0,), start_index_map=(0,))
    return lax.gather(v, idx, dnums, slice_sizes=(1,),
                      mode=lax.GatherScatterMode.PROMISE_IN_BOUNDS)


def _bcast_last(v):
    return _bcast_lane(v, 15)


def _filter_body(src_hbm, dst_hbm, edges_hbm, cnt_hbm,
                 sbuf0, sbuf1, dbuf0, dbuf1, ring, cnt_v,
                 sem_s0, sem_s1, sem_d0, sem_d1):
    wid = lax.axis_index("s") * NC + lax.axis_index("c")
    lo = wid * NPT
    hi = lo + NPT
    out_base = wid * CAP
    iota = lax.iota(jnp.int32, 16)
    sent = ((wid * 16 + iota) << 9) | DSENT

    bufs = [(sbuf0, dbuf0, sem_s0, sem_d0), (sbuf1, dbuf1, sem_s1, sem_d1)]

    def issue(g, p):
        sb, db, ss, ds = bufs[p]
        pltpu.async_copy(src_hbm.at[pl.ds(g * FCH, FCH)], sb, ss)
        pltpu.async_copy(dst_hbm.at[pl.ds(g * FCH, FCH)], db, ds)

    def wait(p):
        sb, db, ss, ds = bufs[p]
        pltpu.make_async_copy(src_hbm.at[pl.ds(0, FCH)], sb, ss).wait()
        pltpu.make_async_copy(dst_hbm.at[pl.ds(0, FCH)], db, ds).wait()

    issue(0, 0)
    issue(1, 1)

    def process_chunk(g, p, carry):
        off, flushed = carry
        sb, db, _, _ = bufs[p]

        def vreg_body(j, off):
            s = sb[pl.ds(j * 16, 16)]
            d = db[pl.ds(j * 16, 16)]
            m = (d >= lo) & (d < hi) & (off < CAP - RING)
            m01 = jnp.where(m, 1, 0).astype(jnp.int32)
            cs = plsc.cumsum(m01)
            pos = (off + cs - 1) & (RING - 1)
            pk = (s << 9) | (d - lo)
            plsc.store_scatter(ring, [pos], pk, mask=m)
            return off + _bcast_last(cs)

        off = plsc.parallel_loop(0, FCH // 16, unroll=4, carry=off)(vreg_body)

        @pl.when(g + 2 < NFCH)
        def _():
            issue(g + 2, p)

        total = jnp.max(off)
        do_flush = total - flushed >= FB

        @pl.when(do_flush)
        def _():
            half = (flushed // FB) % 2
            pltpu.sync_copy(ring.at[pl.ds(pl.multiple_of(half * FB, FB), FB)],
                            edges_hbm.at[pl.ds(pl.multiple_of(out_base + flushed, FB), FB)])

        flushed = jnp.where(do_flush, flushed + FB, flushed)
        return off, flushed

    carry = (jnp.zeros((16,), jnp.int32), jnp.int32(0))

    def outer(i, carry):
        g = i * 2
        wait(0)
        carry = process_chunk(g, 0, carry)
        wait(1)
        carry = process_chunk(g + 1, 1, carry)
        return carry

    off, flushed = lax.fori_loop(0, NFCH // 2, outer, carry)

    def pad_body(k, _):
        pos = (off + k * 16 + iota) & (RING - 1)
        plsc.store_scatter(ring, [pos], sent)
        return 0

    lax.fori_loop(0, GR // 16, pad_body, 0)

    total = jnp.max(off)
    padded = ((total + GR - 1) // GR) * GR

    def drain_cond(fl):
        return fl < padded

    def drain(fl):
        pltpu.sync_copy(ring.at[pl.ds(pl.multiple_of(fl % RING, GR), GR)],
                        edges_hbm.at[pl.ds(pl.multiple_of(out_base + fl, GR), GR)])
        return fl + GR

    lax.while_loop(drain_cond, drain, flushed)

    cnt_v[...] = jnp.zeros((16,), jnp.int32) + padded
    pltpu.sync_copy(cnt_v, cnt_hbm.at[wid])


def _filter_edges(src_pad, dst_pad):
    f = pl.kernel(
        _filter_body,
        out_type=[
            jax.ShapeDtypeStruct((NW * CAP,), jnp.int32),
            jax.ShapeDtypeStruct((NW, 16), jnp.int32),
        ],
        mesh=_mesh,
        scratch_types=[
            pltpu.VMEM((FCH,), jnp.int32),
            pltpu.VMEM((FCH,), jnp.int32),
            pltpu.VMEM((FCH,), jnp.int32),
            pltpu.VMEM((FCH,), jnp.int32),
            pltpu.VMEM((RING,), jnp.int32),
            pltpu.VMEM((16,), jnp.int32),
            pltpu.SemaphoreType.DMA,
            pltpu.SemaphoreType.DMA,
            pltpu.SemaphoreType.DMA,
            pltpu.SemaphoreType.DMA,
        ],
        compiler_params=_sc_params,
    )
    return f(src_pad, dst_pad)


def _agg_body(with_deg, *refs):
    if with_deg:
        (y_hbm, edges_hbm, cnt_hbm, agg_hbm, deg_hbm,
         pk0, pk1, si0, si1, di0, di1, rows0, rows1, agg_v, deg_v,
         cnt_v, sem_p0, sem_p1, sem_g0, sem_g1) = refs
    else:
        (y_hbm, edges_hbm, cnt_hbm, agg_hbm,
         pk0, pk1, si0, si1, di0, di1, rows0, rows1, agg_v,
         cnt_v, sem_p0, sem_p1, sem_g0, sem_g1) = refs
        deg_hbm = deg_v = None

    wid = lax.axis_index("s") * NC + lax.axis_index("c")
    out_base = wid * CAP
    zero16 = jnp.zeros((16,), jnp.float32)
    ones16 = zero16 + 1.0
    iota = lax.iota(jnp.int32, 16)
    col_idx = [iota + c * 16 for c in range(8)]

    def zrow(i, _):
        for c in range(8):
            agg_v[i, pl.ds(c * 16, 16)] = zero16
        return 0

    lax.fori_loop(0, NPT_A, zrow, 0)

    if with_deg:
        def zdeg(i, _):
            deg_v[pl.ds(i * 16, 16)] = zero16
            return 0
        lax.fori_loop(0, NPT_A, zdeg, 0)

    pltpu.sync_copy(cnt_hbm.at[wid], cnt_v)
    n_chunks = jnp.max(cnt_v[...]) >> 7

    bufs = [(pk0, si0, di0, rows0, sem_p0, sem_g0),
            (pk1, si1, di1, rows1, sem_p1, sem_g1)]

    def issue_pk(g, p):
        pk = bufs[p][0]
        sp = bufs[p][4]
        pltpu.async_copy(
            edges_hbm.at[pl.ds(pl.multiple_of(out_base + g * K, K), K)], pk, sp)

    def wait_pk(p):
        pk = bufs[p][0]
        sp = bufs[p][4]
        pltpu.make_async_copy(edges_hbm.at[pl.ds(0, K)], pk, sp).wait()

    def unpack(p):
        pk, si, di = bufs[p][0], bufs[p][1], bufs[p][2]
        for j in range(K // 16):
            v = pk[pl.ds(j * 16, 16)]
            si[pl.ds(j * 16, 16)] = v >> 9
            di[pl.ds(j * 16, 16)] = v & (512 - 1)

    def issue_gather(p):
        si, rows, sg = bufs[p][1], bufs[p][3], bufs[p][5]
        pltpu.async_copy(y_hbm.at[si], rows, sg)

    def wait_gather(p):
        si, rows, sg = bufs[p][1], bufs[p][3], bufs[p][5]
        pltpu.make_async_copy(y_hbm.at[si], rows, sg).wait()

    def accumulate(p):
        di, rows = bufs[p][2], bufs[p][3]

        def blk_body(blk, _):
            dvec = di[pl.ds(blk * 16, 16)]
            for e in range(16):
                db = _bcast_lane(dvec, e)
                ei = blk * 16 + e
                for c in range(8):
                    val = rows[ei, pl.ds(c * 16, 16)]
                    plsc.addupdate_scatter(agg_v, [db, col_idx[c]], val)
                if with_deg:
                    plsc.addupdate_scatter(deg_v, [db * 16 + iota], ones16)
            return 0

        lax.fori_loop(0, K // 16, blk_body, 0)

    @pl.when(n_chunks > 0)
    def _():
        issue_pk(0, 0)

    @pl.when(n_chunks > 1)
    def _():
        issue_pk(1, 1)

    @pl.when(n_chunks > 0)
    def _():
        wait_pk(0)
        unpack(0)
        issue_gather(0)

        @pl.when(n_chunks > 2)
        def _():
            issue_pk(2, 0)

    def chunk(g, p):
        q = 1 - p

        @pl.when(g < n_chunks)
        def _():
            wait_gather(p)

            @pl.when(g + 1 < n_chunks)
            def _():
                wait_pk(q)
                unpack(q)
                issue_gather(q)

                @pl.when(g + 3 < n_chunks)
                def _():
                    issue_pk(g + 3, q)

            accumulate(p)

    def outer(i, _):
        chunk(i * 2, 0)
        chunk(i * 2 + 1, 1)
        return 0

    lax.fori_loop(0, (n_chunks + 1) // 2, outer, 0)

    pltpu.sync_copy(agg_v.at[pl.ds(0, NPT)], agg_hbm.at[pl.ds(wid * NPT, NPT)])
    if with_deg:
        pltpu.sync_copy(deg_v.at[pl.ds(0, NPT * 16)],
                        deg_hbm.at[pl.ds(wid * NPT * 16, NPT * 16)])


def _agg(y, edges_p, cnt, with_deg):
    out_type = [jax.ShapeDtypeStruct((NP, HID), jnp.float32)]
    if with_deg:
        out_type.append(jax.ShapeDtypeStruct((NP * 16,), jnp.float32))
    scratch = [
        pltpu.VMEM((K,), jnp.int32),
        pltpu.VMEM((K,), jnp.int32),
        pltpu.VMEM((K,), jnp.int32),
        pltpu.VMEM((K,), jnp.int32),
        pltpu.VMEM((K,), jnp.int32),
        pltpu.VMEM((K,), jnp.int32),
        pltpu.VMEM((K, HID), jnp.float32),
        pltpu.VMEM((K, HID), jnp.float32),
        pltpu.VMEM((NPT_A, HID), jnp.float32),
    ]
    if with_deg:
        scratch.append(pltpu.VMEM((NPT_A * 16,), jnp.float32))
    scratch.append(pltpu.VMEM((16,), jnp.int32))
    scratch += [pltpu.SemaphoreType.DMA] * 4
    f = pl.kernel(
        functools.partial(_agg_body, with_deg),
        out_type=out_type,
        mesh=_mesh,
        scratch_types=scratch,
        compiler_params=_sc_params,
    )
    return f(y, edges_p, cnt)


def _embed_kernel(h_ref, emb_ref, filt_ref, y_ref):
    h = h_ref[...]
    lanes = jax.lax.broadcasted_iota(jnp.int32, (h.shape[0], 128), 1)
    oh = (h == lanes).astype(jnp.float32)
    x = jnp.dot(oh, emb_ref[...], preferred_element_type=jnp.float32)
    sig = jax.nn.sigmoid(filt_ref[...])
    y_ref[...] = x * sig


def _embed(h_pad, emb, filt_pad):
    return pl.pallas_call(
        _embed_kernel,
        out_shape=jax.ShapeDtypeStruct((NP, HID), jnp.float32),
    )(h_pad, emb, filt_pad)


def _dense_kernel(has_next, y_ref, agg_ref, deg_ref, wt_ref, wb_ref, b_ref,
                  g_ref, beta_ref, w0_ref, b0_ref, w1_ref, b1_ref, w2_ref,
                  b2_ref, nfilt_ref, logits_ref, ynext_ref=None):
    y = y_ref[...]
    deg = jnp.clip(deg_ref[...], 1.0, None)
    mean_n = agg_ref[...] * (1.0 / deg)
    t = (jnp.dot(y, wt_ref[...], preferred_element_type=jnp.float32)
         + jnp.dot(mean_n, wb_ref[...], preferred_element_type=jnp.float32)
         + b_ref[...])
    t = jnp.maximum(t, 0.0)
    rows = jax.lax.broadcasted_iota(jnp.int32, (NP, 1), 0)
    mask = (rows < N).astype(jnp.float32)
    tm = t * mask
    mu = jnp.sum(tm, axis=0, keepdims=True) / N
    ex2 = jnp.sum(tm * tm, axis=0, keepdims=True) / N
    var = ex2 - mu * mu
    tn = (t - mu) * jax.lax.rsqrt(var + 1e-5) * g_ref[...] + beta_ref[...]
    out = y + tn
    z = jnp.maximum(jnp.dot(out, w0_ref[...], preferred_element_type=jnp.float32) + b0_ref[...], 0.0)
    z = jnp.maximum(jnp.dot(z, w1_ref[...], preferred_element_type=jnp.float32) + b1_ref[...], 0.0)
    logits_ref[...] = jnp.dot(z, w2_ref[...], preferred_element_type=jnp.float32) + b2_ref[...]
    if has_next:
        ynext_ref[...] = out * jax.nn.sigmoid(nfilt_ref[...])


def _pad_w(w, rows=HID, cols=HID):
    out = jnp.zeros((rows, cols), jnp.float32)
    return out.at[: w.shape[0], : w.shape[1]].set(w)


def _pad_b(b, cols=HID):
    out = jnp.zeros((1, cols), jnp.float32)
    return out.at[0, : b.shape[0]].set(b)


def _dense_stage(y, agg, deg, lp, mp, next_filt_pad):
    has_next = next_filt_pad is not None
    wt = lp['W'][:HID]
    wb = lp['W'][HID:]
    b = lp['b'][None, :]
    g = lp['gamma'][None, :]
    beta = lp['beta'][None, :]
    w0 = _pad_w(mp['Ws'][0])
    b0 = _pad_b(mp['bs'][0])
    w1 = _pad_w(mp['Ws'][1])
    b1 = _pad_b(mp['bs'][1])
    w2 = _pad_w(mp['Ws'][2])
    b2 = _pad_b(mp['bs'][2])
    nfilt = next_filt_pad if has_next else jnp.zeros((NP, 1), jnp.float32)
    out_shape = [jax.ShapeDtypeStruct((NP, HID), jnp.float32)]
    if has_next:
        out_shape.append(jax.ShapeDtypeStruct((NP, HID), jnp.float32))
    res = pl.pallas_call(
        functools.partial(_dense_kernel, has_next),
        out_shape=out_shape,
    )(y, agg, deg, wt, wb, b, g, beta, w0, b0, w1, b1, w2, b2, nfilt)
    if has_next:
        return res[0], res[1]
    return res[0], None


def kernel(h, edge_index, e, params):
    src = edge_index[0].astype(jnp.int32)
    dst = edge_index[1].astype(jnp.int32)
    src_pad = jnp.pad(src, (0, EPAD - E))
    dst_pad = jnp.pad(dst, (0, EPAD - E), constant_values=BIGDST)
    h_pad = jnp.pad(h.astype(jnp.int32), (0, NP - N)).reshape(NP, 1)

    def pad_filt(f):
        return jnp.pad(f, ((0, NP - N), (0, 0)))

    edges_p, cnt = _filter_edges(src_pad, dst_pad)

    y1 = _embed(h_pad, params['emb'], pad_filt(params['cvx_filter']))
    agg1, deg16 = _agg(y1, edges_p, cnt, True)
    deg_pad = deg16.reshape(NP, 16)[:, :1]
    logits1, y2 = _dense_stage(y1, agg1, deg_pad, params['cvx_layers'][3],
                               params['mlp_cvx'], pad_filt(params['cov_filter']))
    agg2 = _agg(y2, edges_p, cnt, False)[0]
    logits2, y3 = _dense_stage(y2, agg2, deg_pad, params['cov_layers'][3],
                               params['mlp_cov'], pad_filt(params['normal_filter']))
    agg3 = _agg(y3, edges_p, cnt, False)[0]
    logits3, _ = _dense_stage(y3, agg3, deg_pad, params['normal_layers'][3],
                              params['mlp_normal'], None)
    return jnp.concatenate([logits1[:N, :6], logits2[:N, :6], logits3[:N, :6]], axis=0)


# speedup vs baseline: 5.2202x; 1.2938x over previous
import functools

import jax
import jax.numpy as jnp
from jax import lax
from jax.experimental import pallas as pl
from jax.experimental.pallas import tpu as pltpu
from jax.experimental.pallas import tpu_sc as plsc

N = 15279
E = 488928
HID = 128
NP = 15360
NPT = 480
NPT_A = 488
NW = 32
NC = 2
NS = 16

EPAD = 491520
FCH = 4096
NFCH = EPAD // FCH
RING = 8192
FB = 4096
GR = 256
CAP = 32768
K = 128
KA = 256
DSENT = 480
BIGDST = 1 << 20

_mesh = plsc.VectorSubcoreMesh(core_axis_name="c", subcore_axis_name="s")
_sc_params = pltpu.CompilerParams(needs_layout_passes=False)


def _bcast_lane(v, lane):
    idx = jnp.full((16, 1), lane, jnp.int32)
    dnums = lax.GatherDimensionNumbers(
        offset_dims=(), collapsed_slice_dims=(0,), start_index_map=(0,))
    return lax.gather(v, idx, dnums, slice_sizes=(1,),
                      mode=lax.GatherScatterMode.PROMISE_IN_BOUNDS)


def _bcast_last(v):
    return _bcast_lane(v, 15)


def _filter_body(src_hbm, dst_hbm, edges_hbm, cnt_hbm,
                 sbuf0, sbuf1, dbuf0, dbuf1, ring, cnt_v,
                 sem_s0, sem_s1, sem_d0, sem_d1):
    wid = lax.axis_index("s") * NC + lax.axis_index("c")
    lo = wid * NPT
    hi = lo + NPT
    out_base = wid * CAP
    iota = lax.iota(jnp.int32, 16)
    sent = ((wid * 16 + iota) << 9) | DSENT

    bufs = [(sbuf0, dbuf0, sem_s0, sem_d0), (sbuf1, dbuf1, sem_s1, sem_d1)]

    def issue(g, p):
        sb, db, ss, ds = bufs[p]
        pltpu.async_copy(src_hbm.at[pl.ds(g * FCH, FCH)], sb, ss)
        pltpu.async_copy(dst_hbm.at[pl.ds(g * FCH, FCH)], db, ds)

    def wait(p):
        sb, db, ss, ds = bufs[p]
        pltpu.make_async_copy(src_hbm.at[pl.ds(0, FCH)], sb, ss).wait()
        pltpu.make_async_copy(dst_hbm.at[pl.ds(0, FCH)], db, ds).wait()

    issue(0, 0)
    issue(1, 1)

    def process_chunk(g, p, carry):
        off, flushed = carry
        sb, db, _, _ = bufs[p]

        def vreg_body(j, off):
            s = sb[pl.ds(j * 16, 16)]
            d = db[pl.ds(j * 16, 16)]
            m = (d >= lo) & (d < hi) & (off < CAP - RING)
            m01 = jnp.where(m, 1, 0).astype(jnp.int32)
            cs = plsc.cumsum(m01)
            pos = (off + cs - 1) & (RING - 1)
            pk = (s << 9) | (d - lo)
            plsc.store_scatter(ring, [pos], pk, mask=m)
            return off + _bcast_last(cs)

        off = plsc.parallel_loop(0, FCH // 16, unroll=4, carry=off)(vreg_body)

        @pl.when(g + 2 < NFCH)
        def _():
            issue(g + 2, p)

        total = jnp.max(off)
        do_flush = total - flushed >= FB

        @pl.when(do_flush)
        def _():
            half = (flushed // FB) % 2
            pltpu.sync_copy(ring.at[pl.ds(pl.multiple_of(half * FB, FB), FB)],
                            edges_hbm.at[pl.ds(pl.multiple_of(out_base + flushed, FB), FB)])

        flushed = jnp.where(do_flush, flushed + FB, flushed)
        return off, flushed

    carry = (jnp.zeros((16,), jnp.int32), jnp.int32(0))

    def outer(i, carry):
        g = i * 2
        wait(0)
        carry = process_chunk(g, 0, carry)
        wait(1)
        carry = process_chunk(g + 1, 1, carry)
        return carry

    off, flushed = lax.fori_loop(0, NFCH // 2, outer, carry)

    def pad_body(k, _):
        pos = (off + k * 16 + iota) & (RING - 1)
        plsc.store_scatter(ring, [pos], sent)
        return 0

    lax.fori_loop(0, GR // 16, pad_body, 0)

    total = jnp.max(off)
    padded = ((total + GR - 1) // GR) * GR

    def drain_cond(fl):
        return fl < padded

    def drain(fl):
        pltpu.sync_copy(ring.at[pl.ds(pl.multiple_of(fl % RING, GR), GR)],
                        edges_hbm.at[pl.ds(pl.multiple_of(out_base + fl, GR), GR)])
        return fl + GR

    lax.while_loop(drain_cond, drain, flushed)

    cnt_v[...] = jnp.zeros((16,), jnp.int32) + padded
    pltpu.sync_copy(cnt_v, cnt_hbm.at[wid])


def _filter_edges(src_pad, dst_pad):
    f = pl.kernel(
        _filter_body,
        out_type=[
            jax.ShapeDtypeStruct((NW * CAP,), jnp.int32),
            jax.ShapeDtypeStruct((NW, 16), jnp.int32),
        ],
        mesh=_mesh,
        scratch_types=[
            pltpu.VMEM((FCH,), jnp.int32),
            pltpu.VMEM((FCH,), jnp.int32),
            pltpu.VMEM((FCH,), jnp.int32),
            pltpu.VMEM((FCH,), jnp.int32),
            pltpu.VMEM((RING,), jnp.int32),
            pltpu.VMEM((16,), jnp.int32),
            pltpu.SemaphoreType.DMA,
            pltpu.SemaphoreType.DMA,
            pltpu.SemaphoreType.DMA,
            pltpu.SemaphoreType.DMA,
        ],
        compiler_params=_sc_params,
    )
    return f(src_pad, dst_pad)


def _agg_body(*refs):
    (y_hbm, edges_hbm, cnt_hbm, agg_hbm,
     pk0, pk1, si0, si1, di0, di1, rows0, rows1, agg_v,
     cnt_v, sem_p0, sem_p1, sem_g0, sem_g1) = refs

    wid = lax.axis_index("s") * NC + lax.axis_index("c")
    out_base = wid * CAP
    zero16 = jnp.zeros((16,), jnp.float32)
    iota = lax.iota(jnp.int32, 16)
    col_idx = [iota + c * 16 for c in range(8)]

    def zrow(i, _):
        for c in range(8):
            agg_v[i, pl.ds(c * 16, 16)] = zero16
        return 0

    lax.fori_loop(0, NPT_A, zrow, 0)

    pltpu.sync_copy(cnt_hbm.at[wid], cnt_v)
    n_chunks = jnp.max(cnt_v[...]) >> 8

    bufs = [(pk0, si0, di0, rows0, sem_p0, sem_g0),
            (pk1, si1, di1, rows1, sem_p1, sem_g1)]

    def issue_pk(g, p):
        pk = bufs[p][0]
        sp = bufs[p][4]
        pltpu.async_copy(
            edges_hbm.at[pl.ds(pl.multiple_of(out_base + g * KA, KA), KA)], pk, sp)

    def wait_pk(p):
        pk = bufs[p][0]
        sp = bufs[p][4]
        pltpu.make_async_copy(edges_hbm.at[pl.ds(0, KA)], pk, sp).wait()

    def unpack(p):
        pk, si, di = bufs[p][0], bufs[p][1], bufs[p][2]
        for j in range(KA // 16):
            v = pk[pl.ds(j * 16, 16)]
            si[pl.ds(j * 16, 16)] = v >> 9
            di[pl.ds(j * 16, 16)] = v & (512 - 1)

    def issue_gather(p):
        si, rows, sg = bufs[p][1], bufs[p][3], bufs[p][5]
        pltpu.async_copy(y_hbm.at[si], rows, sg)

    def wait_gather(p):
        si, rows, sg = bufs[p][1], bufs[p][3], bufs[p][5]
        pltpu.make_async_copy(y_hbm.at[si], rows, sg).wait()

    def accumulate(p):
        di, rows = bufs[p][2], bufs[p][3]

        def blk_body(blk, _):
            dvec = di[pl.ds(blk * 16, 16)]
            for e in range(16):
                db = _bcast_lane(dvec, e)
                ei = blk * 16 + e
                for c in range(8):
                    val = rows[ei, pl.ds(c * 16, 16)]
                    plsc.addupdate_scatter(agg_v, [db, col_idx[c]], val)
            return 0

        lax.fori_loop(0, KA // 16, blk_body, 0)

    @pl.when(n_chunks > 0)
    def _():
        issue_pk(0, 0)

    @pl.when(n_chunks > 1)
    def _():
        issue_pk(1, 1)

    @pl.when(n_chunks > 0)
    def _():
        wait_pk(0)
        unpack(0)
        issue_gather(0)

        @pl.when(n_chunks > 2)
        def _():
            issue_pk(2, 0)

    def chunk(g, p):
        q = 1 - p

        @pl.when(g < n_chunks)
        def _():
            wait_gather(p)

            @pl.when(g + 1 < n_chunks)
            def _():
                wait_pk(q)
                unpack(q)
                issue_gather(q)

                @pl.when(g + 3 < n_chunks)
                def _():
                    issue_pk(g + 3, q)

            accumulate(p)

    def outer(i, _):
        chunk(i * 2, 0)
        chunk(i * 2 + 1, 1)
        return 0

    lax.fori_loop(0, (n_chunks + 1) // 2, outer, 0)

    pltpu.sync_copy(agg_v.at[pl.ds(0, NPT)], agg_hbm.at[pl.ds(wid * NPT, NPT)])


def _agg(y, edges_p, cnt):
    scratch = [
        pltpu.VMEM((KA,), jnp.int32),
        pltpu.VMEM((KA,), jnp.int32),
        pltpu.VMEM((KA,), jnp.int32),
        pltpu.VMEM((KA,), jnp.int32),
        pltpu.VMEM((KA,), jnp.int32),
        pltpu.VMEM((KA,), jnp.int32),
        pltpu.VMEM((KA, HID), jnp.float32),
        pltpu.VMEM((KA, HID), jnp.float32),
        pltpu.VMEM((NPT_A, HID), jnp.float32),
        pltpu.VMEM((16,), jnp.int32),
    ]
    scratch += [pltpu.SemaphoreType.DMA] * 4
    f = pl.kernel(
        _agg_body,
        out_type=[jax.ShapeDtypeStruct((NP, HID), jnp.float32)],
        mesh=_mesh,
        scratch_types=scratch,
        compiler_params=_sc_params,
    )
    (agg,) = f(y, edges_p, cnt)
    return agg


def _hist_body(h_hbm, sig_hbm, edges_hbm, cnt_hbm, wgt_hbm, deg_hbm,
               h_v, sig_v, pk0, pk1, si0, si1, di0, di1, wgt_v, deg_v,
               cnt_v, sem_p0, sem_p1, sem_t):
    wid = lax.axis_index("s") * NC + lax.axis_index("c")
    out_base = wid * CAP
    zero16 = jnp.zeros((16,), jnp.float32)
    ones16 = zero16 + 1.0
    iota = lax.iota(jnp.int32, 16)
    lane_masks = [iota == e for e in range(16)]

    def zrow(i, _):
        for c in range(8):
            wgt_v[i, pl.ds(c * 16, 16)] = zero16
        return 0

    lax.fori_loop(0, NPT_A, zrow, 0)

    def zdeg(i, _):
        deg_v[pl.ds(i * 16, 16)] = zero16
        return 0

    lax.fori_loop(0, NPT_A, zdeg, 0)

    pltpu.async_copy(h_hbm, h_v, sem_t).wait()
    pltpu.async_copy(sig_hbm, sig_v, sem_t).wait()

    pltpu.sync_copy(cnt_hbm.at[wid], cnt_v)
    n_chunks = jnp.max(cnt_v[...]) >> 7

    bufs = [(pk0, si0, di0, sem_p0), (pk1, si1, di1, sem_p1)]

    def issue_pk(g, p):
        pk, _, _, sp = bufs[p]
        pltpu.async_copy(
            edges_hbm.at[pl.ds(pl.multiple_of(out_base + g * K, K), K)], pk, sp)

    @pl.when(n_chunks > 0)
    def _():
        issue_pk(0, 0)

    @pl.when(n_chunks > 1)
    def _():
        issue_pk(1, 1)

    def chunk(g, p):
        pk, si, di, sp = bufs[p]

        @pl.when(g < n_chunks)
        def _():
            pltpu.make_async_copy(edges_hbm.at[pl.ds(0, K)], pk, sp).wait()
            for j in range(K // 16):
                v = pk[pl.ds(j * 16, 16)]
                si[pl.ds(j * 16, 16)] = v >> 9
                di[pl.ds(j * 16, 16)] = v & (512 - 1)

            @pl.when(g + 2 < n_chunks)
            def _():
                issue_pk(g + 2, p)

            def blk_body(blk, _):
                svec = si[pl.ds(blk * 16, 16)]
                dvec = di[pl.ds(blk * 16, 16)]
                hs = plsc.load_gather(h_v, [svec])
                sg = plsc.load_gather(sig_v, [svec])
                plsc.addupdate_scatter(deg_v, [dvec * 16 + iota], ones16)
                for e in range(16):
                    plsc.addupdate_scatter(wgt_v, [dvec, hs], sg,
                                           mask=lane_masks[e])
                return 0

            lax.fori_loop(0, K // 16, blk_body, 0)

    def outer(i, _):
        chunk(i * 2, 0)
        chunk(i * 2 + 1, 1)
        return 0

    lax.fori_loop(0, (n_chunks + 1) // 2, outer, 0)

    pltpu.sync_copy(wgt_v.at[pl.ds(0, NPT)], wgt_hbm.at[pl.ds(wid * NPT, NPT)])
    pltpu.sync_copy(deg_v.at[pl.ds(0, NPT * 16)],
                    deg_hbm.at[pl.ds(wid * NPT * 16, NPT * 16)])


def _hist(h_flat, sig_flat, edges_p, cnt):
    scratch = [
        pltpu.VMEM((NP,), jnp.int32),
        pltpu.VMEM((NP,), jnp.float32),
        pltpu.VMEM((K,), jnp.int32),
        pltpu.VMEM((K,), jnp.int32),
        pltpu.VMEM((K,), jnp.int32),
        pltpu.VMEM((K,), jnp.int32),
        pltpu.VMEM((K,), jnp.int32),
        pltpu.VMEM((K,), jnp.int32),
        pltpu.VMEM((NPT_A, HID), jnp.float32),
        pltpu.VMEM((NPT_A * 16,), jnp.float32),
        pltpu.VMEM((16,), jnp.int32),
    ]
    scratch += [pltpu.SemaphoreType.DMA] * 3
    f = pl.kernel(
        _hist_body,
        out_type=[
            jax.ShapeDtypeStruct((NP, HID), jnp.float32),
            jax.ShapeDtypeStruct((NP * 16,), jnp.float32),
        ],
        mesh=_mesh,
        scratch_types=scratch,
        compiler_params=_sc_params,
    )
    return f(h_flat, sig_flat, edges_p, cnt)


def _embed_kernel(h_ref, emb_ref, filt_ref, y_ref, sig_ref):
    h = h_ref[...]
    lanes = jax.lax.broadcasted_iota(jnp.int32, (h.shape[0], 128), 1)
    oh = (h == lanes).astype(jnp.float32)
    x = jnp.dot(oh, emb_ref[...], preferred_element_type=jnp.float32)
    sig = jax.nn.sigmoid(filt_ref[...])
    y_ref[...] = x * sig
    sig_ref[...] = sig


def _embed(h_pad, emb, filt_pad):
    return pl.pallas_call(
        _embed_kernel,
        out_shape=[jax.ShapeDtypeStruct((NP, HID), jnp.float32),
                   jax.ShapeDtypeStruct((NP, 1), jnp.float32)],
    )(h_pad, emb, filt_pad)


def _dense_kernel(has_next, from_counts, *refs):
    if from_counts:
        (y_ref, agg_ref, deg_ref, emb_ref, wt_ref, wb_ref, b_ref,
         g_ref, beta_ref, w0_ref, b0_ref, w1_ref, b1_ref, w2_ref,
         b2_ref, nfilt_ref, *outs) = refs
    else:
        (y_ref, agg_ref, deg_ref, wt_ref, wb_ref, b_ref,
         g_ref, beta_ref, w0_ref, b0_ref, w1_ref, b1_ref, w2_ref,
         b2_ref, nfilt_ref, *outs) = refs
        emb_ref = None
    logits_ref = outs[0]
    ynext_ref = outs[1] if has_next else None
    y = y_ref[...]
    deg16 = deg_ref[...]
    deg = jnp.clip(jnp.sum(deg16, axis=1, keepdims=True), 1.0, None)
    agg = agg_ref[...]
    if from_counts:
        agg = jnp.dot(agg, emb_ref[...], preferred_element_type=jnp.float32)
    mean_n = agg * (1.0 / deg)
    t = (jnp.dot(y, wt_ref[...], preferred_element_type=jnp.float32)
         + jnp.dot(mean_n, wb_ref[...], preferred_element_type=jnp.float32)
         + b_ref[...])
    t = jnp.maximum(t, 0.0)
    rows = jax.lax.broadcasted_iota(jnp.int32, (NP, 1), 0)
    mask = (rows < N).astype(jnp.float32)
    tm = t * mask
    mu = jnp.sum(tm, axis=0, keepdims=True) / N
    ex2 = jnp.sum(tm * tm, axis=0, keepdims=True) / N
    var = ex2 - mu * mu
    tn = (t - mu) * jax.lax.rsqrt(var + 1e-5) * g_ref[...] + beta_ref[...]
    out = y + tn
    z = jnp.maximum(jnp.dot(out, w0_ref[...], preferred_element_type=jnp.float32) + b0_ref[...], 0.0)
    z = jnp.maximum(jnp.dot(z, w1_ref[...], preferred_element_type=jnp.float32) + b1_ref[...], 0.0)
    logits_ref[...] = jnp.dot(z, w2_ref[...], preferred_element_type=jnp.float32) + b2_ref[...]
    if has_next:
        ynext_ref[...] = out * jax.nn.sigmoid(nfilt_ref[...])


def _pad_w(w, rows=HID, cols=HID):
    out = jnp.zeros((rows, cols), jnp.float32)
    return out.at[: w.shape[0], : w.shape[1]].set(w)


def _pad_b(b, cols=HID):
    out = jnp.zeros((1, cols), jnp.float32)
    return out.at[0, : b.shape[0]].set(b)


def _dense_stage(y, agg, deg16, lp, mp, next_filt_pad, emb=None):
    has_next = next_filt_pad is not None
    from_counts = emb is not None
    wt = lp['W'][:HID]
    wb = lp['W'][HID:]
    b = lp['b'][None, :]
    g = lp['gamma'][None, :]
    beta = lp['beta'][None, :]
    w0 = _pad_w(mp['Ws'][0])
    b0 = _pad_b(mp['bs'][0])
    w1 = _pad_w(mp['Ws'][1])
    b1 = _pad_b(mp['bs'][1])
    w2 = _pad_w(mp['Ws'][2])
    b2 = _pad_b(mp['bs'][2])
    nfilt = next_filt_pad if has_next else jnp.zeros((NP, 1), jnp.float32)
    out_shape = [jax.ShapeDtypeStruct((NP, HID), jnp.float32)]
    if has_next:
        out_shape.append(jax.ShapeDtypeStruct((NP, HID), jnp.float32))
    args = [y, agg, deg16]
    if from_counts:
        args.append(emb)
    args += [wt, wb, b, g, beta, w0, b0, w1, b1, w2, b2, nfilt]
    res = pl.pallas_call(
        functools.partial(_dense_kernel, has_next, from_counts),
        out_shape=out_shape,
        compiler_params=pltpu.CompilerParams(
            vmem_limit_bytes=100 * 1024 * 1024),
    )(*args)
    if has_next:
        return res[0], res[1]
    return res[0], None


def kernel(h, edge_index, e, params):
    src = edge_index[0].astype(jnp.int32)
    dst = edge_index[1].astype(jnp.int32)
    src_pad = jnp.pad(src, (0, EPAD - E))
    dst_pad = jnp.pad(dst, (0, EPAD - E), constant_values=BIGDST)
    h_flat = jnp.pad(h.astype(jnp.int32), (0, NP - N))
    h_pad = h_flat.reshape(NP, 1)

    def pad_filt(f):
        return jnp.pad(f, ((0, NP - N), (0, 0)))

    edges_p, cnt = _filter_edges(src_pad, dst_pad)

    y1, sig1 = _embed(h_pad, params['emb'], pad_filt(params['cvx_filter']))
    wgt, deg_flat = _hist(h_flat, sig1.reshape(NP), edges_p, cnt)
    deg16 = deg_flat.reshape(NP, 16)
    logits1, y2 = _dense_stage(y1, wgt, deg16, params['cvx_layers'][3],
                               params['mlp_cvx'], pad_filt(params['cov_filter']),
                               emb=params['emb'])
    agg2 = _agg(y2, edges_p, cnt)
    logits2, y3 = _dense_stage(y2, agg2, deg16, params['cov_layers'][3],
                               params['mlp_cov'], pad_filt(params['normal_filter']))
    agg3 = _agg(y3, edges_p, cnt)
    logits3, _ = _dense_stage(y3, agg3, deg16, params['normal_layers'][3],
                              params['mlp_normal'], None)
    return jnp.concatenate([logits1[:N, :6], logits2[:N, :6], logits3[:N, :6]], axis=0)


# speedup vs baseline: 5.2245x; 1.0008x over previous
import functools

import jax
import jax.numpy as jnp
from jax import lax
from jax.experimental import pallas as pl
from jax.experimental.pallas import tpu as pltpu
from jax.experimental.pallas import tpu_sc as plsc

N = 15279
E = 488928
HID = 128
NP = 15360
NPT = 480
NPT_A = 488
NW = 32
NC = 2
NS = 16

EPAD = 491520
FCH = 4096
NFCH = EPAD // FCH
RING = 8192
FB = 4096
GR = 256
CAP = 32768
K = 128
KA = 256
DSENT = 480
BIGDST = 1 << 20

_mesh = plsc.VectorSubcoreMesh(core_axis_name="c", subcore_axis_name="s")
_sc_params = pltpu.CompilerParams(needs_layout_passes=False)


def _bcast_lane(v, lane):
    idx = jnp.full((16, 1), lane, jnp.int32)
    dnums = lax.GatherDimensionNumbers(
        offset_dims=(), collapsed_slice_dims=(0,), start_index_map=(0,))
    return lax.gather(v, idx, dnums, slice_sizes=(1,),
                      mode=lax.GatherScatterMode.PROMISE_IN_BOUNDS)


def _bcast_last(v):
    return _bcast_lane(v, 15)


def _filter_body(src_hbm, dst_hbm, edges_hbm, cnt_hbm,
                 sbuf0, sbuf1, dbuf0, dbuf1, ring, cnt_v,
                 sem_s0, sem_s1, sem_d0, sem_d1):
    wid = lax.axis_index("s") * NC + lax.axis_index("c")
    lo = wid * NPT
    hi = lo + NPT
    out_base = wid * CAP
    iota = lax.iota(jnp.int32, 16)
    sent = ((wid * 16 + iota) << 9) | DSENT

    bufs = [(sbuf0, dbuf0, sem_s0, sem_d0), (sbuf1, dbuf1, sem_s1, sem_d1)]

    def issue(g, p):
        sb, db, ss, ds = bufs[p]
        pltpu.async_copy(src_hbm.at[pl.ds(g * FCH, FCH)], sb, ss)
        pltpu.async_copy(dst_hbm.at[pl.ds(g * FCH, FCH)], db, ds)

    def wait(p):
        sb, db, ss, ds = bufs[p]
        pltpu.make_async_copy(src_hbm.at[pl.ds(0, FCH)], sb, ss).wait()
        pltpu.make_async_copy(dst_hbm.at[pl.ds(0, FCH)], db, ds).wait()

    issue(0, 0)
    issue(1, 1)

    def process_chunk(g, p, carry):
        off, flushed = carry
        sb, db, _, _ = bufs[p]

        def vreg_body(j, off):
            s = sb[pl.ds(j * 16, 16)]
            d = db[pl.ds(j * 16, 16)]
            m = (d >= lo) & (d < hi) & (off < CAP - RING)
            m01 = jnp.where(m, 1, 0).astype(jnp.int32)
            cs = plsc.cumsum(m01)
            pos = (off + cs - 1) & (RING - 1)
            pk = (s << 9) | (d - lo)
            plsc.store_scatter(ring, [pos], pk, mask=m)
            return off + _bcast_last(cs)

        off = plsc.parallel_loop(0, FCH // 16, unroll=4, carry=off)(vreg_body)

        @pl.when(g + 2 < NFCH)
        def _():
            issue(g + 2, p)

        total = jnp.max(off)
        do_flush = total - flushed >= FB

        @pl.when(do_flush)
        def _():
            half = (flushed // FB) % 2
            pltpu.sync_copy(ring.at[pl.ds(pl.multiple_of(half * FB, FB), FB)],
                            edges_hbm.at[pl.ds(pl.multiple_of(out_base + flushed, FB), FB)])

        flushed = jnp.where(do_flush, flushed + FB, flushed)
        return off, flushed

    carry = (jnp.zeros((16,), jnp.int32), jnp.int32(0))

    def outer(i, carry):
        g = i * 2
        wait(0)
        carry = process_chunk(g, 0, carry)
        wait(1)
        carry = process_chunk(g + 1, 1, carry)
        return carry

    off, flushed = lax.fori_loop(0, NFCH // 2, outer, carry)

    def pad_body(k, _):
        pos = (off + k * 16 + iota) & (RING - 1)
        plsc.store_scatter(ring, [pos], sent)
        return 0

    lax.fori_loop(0, GR // 16, pad_body, 0)

    total = jnp.max(off)
    padded = ((total + GR - 1) // GR) * GR

    def drain_cond(fl):
        return fl < padded

    def drain(fl):
        pltpu.sync_copy(ring.at[pl.ds(pl.multiple_of(fl % RING, GR), GR)],
                        edges_hbm.at[pl.ds(pl.multiple_of(out_base + fl, GR), GR)])
        return fl + GR

    lax.while_loop(drain_cond, drain, flushed)

    cnt_v[...] = jnp.zeros((16,), jnp.int32) + padded
    pltpu.sync_copy(cnt_v, cnt_hbm.at[wid])


def _filter_edges(src_pad, dst_pad):
    f = pl.kernel(
        _filter_body,
        out_type=[
            jax.ShapeDtypeStruct((NW * CAP,), jnp.int32),
            jax.ShapeDtypeStruct((NW, 16), jnp.int32),
        ],
        mesh=_mesh,
        scratch_types=[
            pltpu.VMEM((FCH,), jnp.int32),
            pltpu.VMEM((FCH,), jnp.int32),
            pltpu.VMEM((FCH,), jnp.int32),
            pltpu.VMEM((FCH,), jnp.int32),
            pltpu.VMEM((RING,), jnp.int32),
            pltpu.VMEM((16,), jnp.int32),
            pltpu.SemaphoreType.DMA,
            pltpu.SemaphoreType.DMA,
            pltpu.SemaphoreType.DMA,
            pltpu.SemaphoreType.DMA,
        ],
        compiler_params=_sc_params,
    )
    return f(src_pad, dst_pad)


def _agg_body(*refs):
    (y_hbm, edges_hbm, cnt_hbm, agg_hbm,
     pk0, pk1, si0, si1, di0, di1, rows0, rows1, agg_v,
     cnt_v, sem_p0, sem_p1, sem_g0, sem_g1) = refs

    wid = lax.axis_index("s") * NC + lax.axis_index("c")
    out_base = wid * CAP
    zero16 = jnp.zeros((16,), jnp.float32)
    iota = lax.iota(jnp.int32, 16)
    col_idx = [iota + c * 16 for c in range(8)]

    def zrow(i, _):
        for c in range(8):
            agg_v[i, pl.ds(c * 16, 16)] = zero16
        return 0

    lax.fori_loop(0, NPT_A, zrow, 0)

    pltpu.sync_copy(cnt_hbm.at[wid], cnt_v)
    n_chunks = jnp.max(cnt_v[...]) >> 8

    bufs = [(pk0, si0, di0, rows0, sem_p0, sem_g0),
            (pk1, si1, di1, rows1, sem_p1, sem_g1)]

    def issue_pk(g, p):
        pk = bufs[p][0]
        sp = bufs[p][4]
        pltpu.async_copy(
            edges_hbm.at[pl.ds(pl.multiple_of(out_base + g * KA, KA), KA)], pk, sp)

    def wait_pk(p):
        pk = bufs[p][0]
        sp = bufs[p][4]
        pltpu.make_async_copy(edges_hbm.at[pl.ds(0, KA)], pk, sp).wait()

    def unpack(p):
        pk, si, di = bufs[p][0], bufs[p][1], bufs[p][2]
        for j in range(KA // 16):
            v = pk[pl.ds(j * 16, 16)]
            si[pl.ds(j * 16, 16)] = v >> 9
            di[pl.ds(j * 16, 16)] = v & (512 - 1)

    def issue_gather(p):
        si, rows, sg = bufs[p][1], bufs[p][3], bufs[p][5]
        pltpu.async_copy(y_hbm.at[si.at[pl.ds(0, KA // 2)]],
                         rows.at[pl.ds(0, KA // 2)], sg)
        pltpu.async_copy(y_hbm.at[si.at[pl.ds(KA // 2, KA // 2)]],
                         rows.at[pl.ds(KA // 2, KA // 2)], sg)

    def wait_gather(p):
        si, rows, sg = bufs[p][1], bufs[p][3], bufs[p][5]
        pltpu.make_async_copy(y_hbm.at[si.at[pl.ds(0, KA // 2)]],
                              rows.at[pl.ds(0, KA // 2)], sg).wait()
        pltpu.make_async_copy(y_hbm.at[si.at[pl.ds(KA // 2, KA // 2)]],
                              rows.at[pl.ds(KA // 2, KA // 2)], sg).wait()

    def accumulate(p):
        di, rows = bufs[p][2], bufs[p][3]

        def blk_body(blk, _):
            dvec = di[pl.ds(blk * 16, 16)]
            for e in range(16):
                db = _bcast_lane(dvec, e)
                ei = blk * 16 + e
                for c in range(8):
                    val = rows[ei, pl.ds(c * 16, 16)]
                    plsc.addupdate_scatter(agg_v, [db, col_idx[c]], val)
            return 0

        lax.fori_loop(0, KA // 16, blk_body, 0)

    @pl.when(n_chunks > 0)
    def _():
        issue_pk(0, 0)

    @pl.when(n_chunks > 1)
    def _():
        issue_pk(1, 1)

    @pl.when(n_chunks > 0)
    def _():
        wait_pk(0)
        unpack(0)
        issue_gather(0)

        @pl.when(n_chunks > 2)
        def _():
            issue_pk(2, 0)

    def chunk(g, p):
        q = 1 - p

        @pl.when(g < n_chunks)
        def _():
            wait_gather(p)

            @pl.when(g + 1 < n_chunks)
            def _():
                wait_pk(q)
                unpack(q)
                issue_gather(q)

                @pl.when(g + 3 < n_chunks)
                def _():
                    issue_pk(g + 3, q)

            accumulate(p)

    def outer(i, _):
        chunk(i * 2, 0)
        chunk(i * 2 + 1, 1)
        return 0

    lax.fori_loop(0, (n_chunks + 1) // 2, outer, 0)

    pltpu.sync_copy(agg_v.at[pl.ds(0, NPT)], agg_hbm.at[pl.ds(wid * NPT, NPT)])


def _agg(y, edges_p, cnt):
    scratch = [
        pltpu.VMEM((KA,), jnp.int32),
        pltpu.VMEM((KA,), jnp.int32),
        pltpu.VMEM((KA,), jnp.int32),
        pltpu.VMEM((KA,), jnp.int32),
        pltpu.VMEM((KA,), jnp.int32),
        pltpu.VMEM((KA,), jnp.int32),
        pltpu.VMEM((KA, HID), jnp.float32),
        pltpu.VMEM((KA, HID), jnp.float32),
        pltpu.VMEM((NPT_A, HID), jnp.float32),
        pltpu.VMEM((16,), jnp.int32),
    ]
    scratch += [pltpu.SemaphoreType.DMA] * 4
    f = pl.kernel(
        _agg_body,
        out_type=[jax.ShapeDtypeStruct((NP, HID), jnp.float32)],
        mesh=_mesh,
        scratch_types=scratch,
        compiler_params=_sc_params,
    )
    (agg,) = f(y, edges_p, cnt)
    return agg


def _hist_body(h_hbm, sig_hbm, edges_hbm, cnt_hbm, wgt_hbm, deg_hbm,
               h_v, sig_v, pk0, pk1, si0, si1, di0, di1, wgt_v, deg_v,
               cnt_v, sem_p0, sem_p1, sem_t):
    wid = lax.axis_index("s") * NC + lax.axis_index("c")
    out_base = wid * CAP
    zero16 = jnp.zeros((16,), jnp.float32)
    ones16 = zero16 + 1.0
    iota = lax.iota(jnp.int32, 16)
    lane_masks = [iota == e for e in range(16)]

    def zrow(i, _):
        for c in range(8):
            wgt_v[i, pl.ds(c * 16, 16)] = zero16
        return 0

    lax.fori_loop(0, NPT_A, zrow, 0)

    def zdeg(i, _):
        deg_v[pl.ds(i * 16, 16)] = zero16
        return 0

    lax.fori_loop(0, NPT_A, zdeg, 0)

    pltpu.async_copy(h_hbm, h_v, sem_t).wait()
    pltpu.async_copy(sig_hbm, sig_v, sem_t).wait()

    pltpu.sync_copy(cnt_hbm.at[wid], cnt_v)
    n_chunks = jnp.max(cnt_v[...]) >> 7

    bufs = [(pk0, si0, di0, sem_p0), (pk1, si1, di1, sem_p1)]

    def issue_pk(g, p):
        pk, _, _, sp = bufs[p]
        pltpu.async_copy(
            edges_hbm.at[pl.ds(pl.multiple_of(out_base + g * K, K), K)], pk, sp)

    @pl.when(n_chunks > 0)
    def _():
        issue_pk(0, 0)

    @pl.when(n_chunks > 1)
    def _():
        issue_pk(1, 1)

    def chunk(g, p):
        pk, si, di, sp = bufs[p]

        @pl.when(g < n_chunks)
        def _():
            pltpu.make_async_copy(edges_hbm.at[pl.ds(0, K)], pk, sp).wait()
            for j in range(K // 16):
                v = pk[pl.ds(j * 16, 16)]
                si[pl.ds(j * 16, 16)] = v >> 9
                di[pl.ds(j * 16, 16)] = v & (512 - 1)

            @pl.when(g + 2 < n_chunks)
            def _():
                issue_pk(g + 2, p)

            def blk_body(blk, _):
                svec = si[pl.ds(blk * 16, 16)]
                dvec = di[pl.ds(blk * 16, 16)]
                hs = plsc.load_gather(h_v, [svec])
                sg = plsc.load_gather(sig_v, [svec])
                plsc.addupdate_scatter(deg_v, [dvec * 16 + iota], ones16)
                for e in range(16):
                    plsc.addupdate_scatter(wgt_v, [dvec, hs], sg,
                                           mask=lane_masks[e])
                return 0

            lax.fori_loop(0, K // 16, blk_body, 0)

    def outer(i, _):
        chunk(i * 2, 0)
        chunk(i * 2 + 1, 1)
        return 0

    lax.fori_loop(0, (n_chunks + 1) // 2, outer, 0)

    pltpu.sync_copy(wgt_v.at[pl.ds(0, NPT)], wgt_hbm.at[pl.ds(wid * NPT, NPT)])
    pltpu.sync_copy(deg_v.at[pl.ds(0, NPT * 16)],
                    deg_hbm.at[pl.ds(wid * NPT * 16, NPT * 16)])


def _hist(h_flat, sig_flat, edges_p, cnt):
    scratch = [
        pltpu.VMEM((NP,), jnp.int32),
        pltpu.VMEM((NP,), jnp.float32),
        pltpu.VMEM((K,), jnp.int32),
        pltpu.VMEM((K,), jnp.int32),
        pltpu.VMEM((K,), jnp.int32),
        pltpu.VMEM((K,), jnp.int32),
        pltpu.VMEM((K,), jnp.int32),
        pltpu.VMEM((K,), jnp.int32),
        pltpu.VMEM((NPT_A, HID), jnp.float32),
        pltpu.VMEM((NPT_A * 16,), jnp.float32),
        pltpu.VMEM((16,), jnp.int32),
    ]
    scratch += [pltpu.SemaphoreType.DMA] * 3
    f = pl.kernel(
        _hist_body,
        out_type=[
            jax.ShapeDtypeStruct((NP, HID), jnp.float32),
            jax.ShapeDtypeStruct((NP * 16,), jnp.float32),
        ],
        mesh=_mesh,
        scratch_types=scratch,
        compiler_params=_sc_params,
    )
    return f(h_flat, sig_flat, edges_p, cnt)


def _embed_kernel(h_ref, emb_ref, filt_ref, y_ref, sig_ref):
    h = h_ref[...]
    lanes = jax.lax.broadcasted_iota(jnp.int32, (h.shape[0], 128), 1)
    oh = (h == lanes).astype(jnp.float32)
    x = jnp.dot(oh, emb_ref[...], preferred_element_type=jnp.float32)
    sig = jax.nn.sigmoid(filt_ref[...])
    y_ref[...] = x * sig
    sig_ref[...] = sig


def _embed(h_pad, emb, filt_pad):
    return pl.pallas_call(
        _embed_kernel,
        out_shape=[jax.ShapeDtypeStruct((NP, HID), jnp.float32),
                   jax.ShapeDtypeStruct((NP, 1), jnp.float32)],
    )(h_pad, emb, filt_pad)


def _dense_kernel(has_next, from_counts, *refs):
    if from_counts:
        (y_ref, agg_ref, deg_ref, emb_ref, wt_ref, wb_ref, b_ref,
         g_ref, beta_ref, w0_ref, b0_ref, w1_ref, b1_ref, w2_ref,
         b2_ref, nfilt_ref, *outs) = refs
    else:
        (y_ref, agg_ref, deg_ref, wt_ref, wb_ref, b_ref,
         g_ref, beta_ref, w0_ref, b0_ref, w1_ref, b1_ref, w2_ref,
         b2_ref, nfilt_ref, *outs) = refs
        emb_ref = None
    logits_ref = outs[0]
    ynext_ref = outs[1] if has_next else None
    y = y_ref[...]
    deg16 = deg_ref[...]
    deg = jnp.clip(jnp.sum(deg16, axis=1, keepdims=True), 1.0, None)
    agg = agg_ref[...]
    if from_counts:
        agg = jnp.dot(agg, emb_ref[...], preferred_element_type=jnp.float32)
    mean_n = agg * (1.0 / deg)
    t = (jnp.dot(y, wt_ref[...], preferred_element_type=jnp.float32)
         + jnp.dot(mean_n, wb_ref[...], preferred_element_type=jnp.float32)
         + b_ref[...])
    t = jnp.maximum(t, 0.0)
    rows = jax.lax.broadcasted_iota(jnp.int32, (NP, 1), 0)
    mask = (rows < N).astype(jnp.float32)
    tm = t * mask
    mu = jnp.sum(tm, axis=0, keepdims=True) / N
    ex2 = jnp.sum(tm * tm, axis=0, keepdims=True) / N
    var = ex2 - mu * mu
    tn = (t - mu) * jax.lax.rsqrt(var + 1e-5) * g_ref[...] + beta_ref[...]
    out = y + tn
    z = jnp.maximum(jnp.dot(out, w0_ref[...], preferred_element_type=jnp.float32) + b0_ref[...], 0.0)
    z = jnp.maximum(jnp.dot(z, w1_ref[...], preferred_element_type=jnp.float32) + b1_ref[...], 0.0)
    logits_ref[...] = jnp.dot(z, w2_ref[...], preferred_element_type=jnp.float32) + b2_ref[...]
    if has_next:
        ynext_ref[...] = out * jax.nn.sigmoid(nfilt_ref[...])


def _pad_w(w, rows=HID, cols=HID):
    out = jnp.zeros((rows, cols), jnp.float32)
    return out.at[: w.shape[0], : w.shape[1]].set(w)


def _pad_b(b, cols=HID):
    out = jnp.zeros((1, cols), jnp.float32)
    return out.at[0, : b.shape[0]].set(b)


def _dense_stage(y, agg, deg16, lp, mp, next_filt_pad, emb=None):
    has_next = next_filt_pad is not None
    from_counts = emb is not None
    wt = lp['W'][:HID]
    wb = lp['W'][HID:]
    b = lp['b'][None, :]
    g = lp['gamma'][None, :]
    beta = lp['beta'][None, :]
    w0 = _pad_w(mp['Ws'][0])
    b0 = _pad_b(mp['bs'][0])
    w1 = _pad_w(mp['Ws'][1])
    b1 = _pad_b(mp['bs'][1])
    w2 = _pad_w(mp['Ws'][2])
    b2 = _pad_b(mp['bs'][2])
    nfilt = next_filt_pad if has_next else jnp.zeros((NP, 1), jnp.float32)
    out_shape = [jax.ShapeDtypeStruct((NP, HID), jnp.float32)]
    if has_next:
        out_shape.append(jax.ShapeDtypeStruct((NP, HID), jnp.float32))
    args = [y, agg, deg16]
    if from_counts:
        args.append(emb)
    args += [wt, wb, b, g, beta, w0, b0, w1, b1, w2, b2, nfilt]
    res = pl.pallas_call(
        functools.partial(_dense_kernel, has_next, from_counts),
        out_shape=out_shape,
        compiler_params=pltpu.CompilerParams(
            vmem_limit_bytes=100 * 1024 * 1024),
    )(*args)
    if has_next:
        return res[0], res[1]
    return res[0], None


def kernel(h, edge_index, e, params):
    src = edge_index[0].astype(jnp.int32)
    dst = edge_index[1].astype(jnp.int32)
    src_pad = jnp.pad(src, (0, EPAD - E))
    dst_pad = jnp.pad(dst, (0, EPAD - E), constant_values=BIGDST)
    h_flat = jnp.pad(h.astype(jnp.int32), (0, NP - N))
    h_pad = h_flat.reshape(NP, 1)

    def pad_filt(f):
        return jnp.pad(f, ((0, NP - N), (0, 0)))

    edges_p, cnt = _filter_edges(src_pad, dst_pad)

    y1, sig1 = _embed(h_pad, params['emb'], pad_filt(params['cvx_filter']))
    wgt, deg_flat = _hist(h_flat, sig1.reshape(NP), edges_p, cnt)
    deg16 = deg_flat.reshape(NP, 16)
    logits1, y2 = _dense_stage(y1, wgt, deg16, params['cvx_layers'][3],
                               params['mlp_cvx'], pad_filt(params['cov_filter']),
                               emb=params['emb'])
    agg2 = _agg(y2, edges_p, cnt)
    logits2, y3 = _dense_stage(y2, agg2, deg16, params['cov_layers'][3],
                               params['mlp_cov'], pad_filt(params['normal_filter']))
    agg3 = _agg(y3, edges_p, cnt)
    logits3, _ = _dense_stage(y3, agg3, deg16, params['normal_layers'][3],
                              params['mlp_normal'], None)
    return jnp.concatenate([logits1[:N, :6], logits2[:N, :6], logits3[:N, :6]], axis=0)
